# baseline (speedup 1.0000x reference)
import functools

import jax
import jax.numpy as jnp
from jax import lax
from jax.experimental import pallas as pl
from jax.experimental.pallas import tpu as pltpu
from jax.experimental.pallas import tpu_sc as plsc

V = 100000
D = 100
DP = 128
GW = 200
NC, NS = 2, 16
NW = NC * NS
XR = 4096
NG = XR // NW
REPACK_COLS = (0, 16, 32, 48, 64, 80, 84)
PC = 1024


def _pad_body(wt_ref, o_ref):
    o_ref[:, :D] = wt_ref[...].T
    o_ref[:, D:] = jnp.zeros((PC, DP - D), jnp.float32)


def _pad_table(wt):
    return pl.pallas_call(
        _pad_body,
        grid=((V + PC - 1) // PC,),
        in_specs=[pl.BlockSpec((D, PC), lambda i: (0, i))],
        out_specs=pl.BlockSpec((PC, DP), lambda i: (i, 0)),
        out_shape=jax.ShapeDtypeStruct((V, DP), jnp.float32),
    )(wt)


def _emb_body(idx_hbm, tab_hbm, out_hbm, idx_v, rows_v, packed_v,
              isem, gsem, ssem):
    wid = lax.axis_index("s") * NC + lax.axis_index("c")
    row0 = wid * NG

    def idx_row(g):
        return jnp.minimum(row0 + g, XR - 1)

    def fire_idx(g, b):
        pltpu.async_copy(
            idx_hbm.at[pl.ds(idx_row(g), 1)], idx_v.at[pl.ds(b, 1)], isem)

    def wait_idx(g, b):
        pltpu.make_async_copy(
            idx_hbm.at[pl.ds(idx_row(g), 1)], idx_v.at[pl.ds(b, 1)],
            isem).wait()

    def fire_gather(b):
        pltpu.async_copy(tab_hbm.at[idx_v.at[b, pl.ds(0, 128)]],
                         rows_v.at[b, pl.ds(0, 128)], gsem)
        pltpu.async_copy(tab_hbm.at[idx_v.at[b, pl.ds(128, GW - 128)]],
                         rows_v.at[b, pl.ds(128, GW - 128)], gsem)

    def wait_gather(b):
        pltpu.make_async_copy(
            tab_hbm.at[idx_v.at[b, pl.ds(0, 128)]],
            rows_v.at[b, pl.ds(0, 128)], gsem).wait()
        pltpu.make_async_copy(
            tab_hbm.at[idx_v.at[b, pl.ds(128, GW - 128)]],
            rows_v.at[b, pl.ds(128, GW - 128)], gsem).wait()

    def fire_store(g, b):
        pltpu.async_copy(
            packed_v.at[b], out_hbm.at[pl.ds((row0 + g) * GW, GW)], ssem)

    def wait_store(g, b):
        pltpu.make_async_copy(
            packed_v.at[b], out_hbm.at[pl.ds((row0 + g) * GW, GW)],
            ssem).wait()

    def repack(b):
        def body(r, carry):
            for u in range(4):
                for c in REPACK_COLS:
                    packed_v[b, 4 * r + u, pl.ds(c, 16)] = \
                        rows_v[b, 4 * r + u, pl.ds(c, 16)]
            return carry
        lax.fori_loop(0, GW // 4, body, 0)

    def step(g, b, drain_store):
        wait_gather(b)
        if drain_store:
            wait_store(g, b)
        repack(b)
        fire_store(g, b)
        wait_idx(g + 1, 1 - b)
        fire_gather(1 - b)
        fire_idx(g + 2, b)

    fire_idx(0, 0)
    wait_idx(0, 0)
    fire_gather(0)
    fire_idx(1, 1)
    step(0, 0, False)
    step(1, 1, False)

    def body(h, carry):
        step(2 * h, 0, True)
        step(2 * h + 1, 1, True)
        return carry

    lax.fori_loop(1, NG // 2, body, 0)

    wait_gather(0)
    wait_idx(NG + 1, 0)
    wait_store(NG - 2, 0)
    wait_store(NG - 1, 1)


@jax.jit
def kernel(x, weight):
    idx = x.astype(jnp.int32)
    wpad = _pad_table(weight.T)
    mesh = plsc.VectorSubcoreMesh(core_axis_name="c", subcore_axis_name="s")
    out = pl.kernel(
        _emb_body,
        mesh=mesh,
        compiler_params=pltpu.CompilerParams(use_tc_tiling_on_sc=False),
        out_type=jax.ShapeDtypeStruct((XR * GW, D), jnp.float32),
        scratch_types=[
            pltpu.VMEM((2, GW), jnp.int32),
            pltpu.VMEM((2, GW, DP), jnp.float32),
            pltpu.VMEM((2, GW, D), jnp.float32),
            pltpu.SemaphoreType.DMA,
            pltpu.SemaphoreType.DMA,
            pltpu.SemaphoreType.DMA,
        ],
    )(idx, wpad)
    return out.reshape(XR, GW, D)


# speedup vs baseline: 2.3388x; 2.3388x over previous
import functools

import jax
import jax.numpy as jnp
from jax import lax
from jax.experimental import pallas as pl
from jax.experimental.pallas import tpu as pltpu
from jax.experimental.pallas import tpu_sc as plsc

V = 100000
D = 100
DP = 128
GW = 200
NC, NS = 2, 16
NW = NC * NS
XR = 4096
NG = XR // NW
REPACK_COLS = (0, 16, 32, 48, 64, 80, 84)
PC = 1024


def _pad_body(wt_ref, o_ref):
    o_ref[:, :D] = wt_ref[...].T
    o_ref[:, D:] = jnp.zeros((PC, DP - D), jnp.float32)


def _pad_table(wt):
    return pl.pallas_call(
        _pad_body,
        grid=((V + PC - 1) // PC,),
        in_specs=[pl.BlockSpec((D, PC), lambda i: (0, i))],
        out_specs=pl.BlockSpec((PC, DP), lambda i: (i, 0)),
        out_shape=jax.ShapeDtypeStruct((V, DP), jnp.float32),
    )(wt)


def _emb_body(idx_hbm, tab_hbm, out_hbm, idx_v, rows_v, packed_v,
              isem, gsem, ssem):
    wid = lax.axis_index("s") * NC + lax.axis_index("c")
    row0 = wid * NG

    def idx_row(g):
        return jnp.minimum(row0 + g, XR - 1)

    def fire_idx(g, b):
        pltpu.async_copy(
            idx_hbm.at[pl.ds(idx_row(g), 1)], idx_v.at[pl.ds(b, 1)], isem)

    def wait_idx(g, b):
        pltpu.make_async_copy(
            idx_hbm.at[pl.ds(idx_row(g), 1)], idx_v.at[pl.ds(b, 1)],
            isem).wait()

    def fire_gather(b):
        pltpu.async_copy(tab_hbm.at[idx_v.at[b, pl.ds(0, 128)]],
                         rows_v.at[b, pl.ds(0, 128)], gsem)
        pltpu.async_copy(tab_hbm.at[idx_v.at[b, pl.ds(128, GW - 128)]],
                         rows_v.at[b, pl.ds(128, GW - 128)], gsem)

    def wait_gather(b):
        pltpu.make_async_copy(
            tab_hbm.at[idx_v.at[b, pl.ds(0, 128)]],
            rows_v.at[b, pl.ds(0, 128)], gsem).wait()
        pltpu.make_async_copy(
            tab_hbm.at[idx_v.at[b, pl.ds(128, GW - 128)]],
            rows_v.at[b, pl.ds(128, GW - 128)], gsem).wait()

    def fire_store(g, b):
        pltpu.async_copy(
            packed_v.at[b], out_hbm.at[pl.ds((row0 + g) * GW, GW)], ssem)

    def wait_store(g, b):
        pltpu.make_async_copy(
            packed_v.at[b], out_hbm.at[pl.ds((row0 + g) * GW, GW)],
            ssem).wait()

    def repack(b):
        def body(r, carry):
            for u in range(4):
                for c in REPACK_COLS:
                    packed_v[b, 4 * r + u, pl.ds(c, 16)] = \
                        rows_v[b, 4 * r + u, pl.ds(c, 16)]
            return carry
        lax.fori_loop(0, GW // 4, body, 0)

    def step(g, b, drain_store):
        wait_gather(b)
        if drain_store:
            wait_store(g, b)
        repack(b)
        fire_store(g, b)
        wait_idx(g + 1, 1 - b)
        fire_gather(1 - b)
        fire_idx(g + 2, b)

    fire_idx(0, 0)
    wait_idx(0, 0)
    fire_gather(0)
    fire_idx(1, 1)
    step(0, 0, False)
    step(1, 1, False)

    def body(h, carry):
        step(2 * h, 0, True)
        step(2 * h + 1, 1, True)
        return carry

    lax.fori_loop(1, NG // 2, body, 0)

    wait_gather(0)
    wait_idx(NG + 1, 0)
    wait_store(NG - 2, 0)
    wait_store(NG - 1, 1)


@jax.jit
def kernel(x, weight):
    idx = x.astype(jnp.int32)
    wpad = _pad_table(weight.T)
    mesh = plsc.VectorSubcoreMesh(core_axis_name="c", subcore_axis_name="s")
    out = pl.kernel(
        _emb_body,
        mesh=mesh,
        out_type=jax.ShapeDtypeStruct((XR * GW, D), jnp.float32),
        scratch_types=[
            pltpu.VMEM((2, GW), jnp.int32),
            pltpu.VMEM((2, GW, DP), jnp.float32),
            pltpu.VMEM((2, GW, D), jnp.float32),
            pltpu.SemaphoreType.DMA,
            pltpu.SemaphoreType.DMA,
            pltpu.SemaphoreType.DMA,
        ],
    )(idx, wpad)
    return out.reshape(XR, GW, D)
